# Initial kernel scaffold; baseline (speedup 1.0000x reference)
import jax
import jax.numpy as jnp
from jax.experimental import pallas as pl


def kernel(alt_flat, nonartifact_centroid_e, nonartifact_log_stdev, artifact_directions_ke, mu_k, log_sigma_k, log_lambda_k, artifact_log_stdev_k, cluster_weights_pre_softmax_k, segment_ids):
    raise NotImplementedError("write your pallas kernel here")


# speedup vs baseline: 2.8918x; 2.8918x over previous
import functools

import jax
import jax.numpy as jnp
from jax.experimental import pallas as pl
from jax.experimental.pallas import tpu as pltpu

_LOG_TWO = 0.6931471805599453
_HIGHEST = jax.lax.Precision.HIGHEST

_ERFC_P = (2.326819970068386e-2, -1.387039388740657e-1, 3.687424674597105e-1,
           -5.824733027278666e-1, 6.210004621745983e-1, -4.944515323274145e-1,
           3.404879937665872e-1, -2.741127028184656e-1, 5.638259427386472e-1)
_ERFC_R = (-1.047766399936249e+1, 1.297719955372516e+1, -7.495518717768503e+0,
           2.921019019210786e+0, -1.015265279202700e+0, 4.218463358204948e-1,
           -2.820767439740514e-1, 5.641895067754075e-1)
_ERF_T = (7.853861353153693e-5, -8.010193625184903e-4, 5.188327685732524e-3,
          -2.685381193529856e-2, 1.128358514861418e-1, -3.761262582423300e-1,
          1.128379165726710e+0)


def _poly(y, coeffs):
    acc = jnp.full_like(y, coeffs[0])
    for c in coeffs[1:]:
        acc = acc * y + c
    return acc


def _erfc(x):
    abs_x = jnp.abs(x)
    z = jnp.exp(-x * x)
    q = 1.0 / abs_x
    y = q * q
    p = jnp.where(abs_x < 2.0, _poly(y, _ERFC_P), _poly(y, _ERFC_R))
    y = z * q * p
    erfc_large = jnp.where(x < 0.0, 2.0 - y, y)
    erf_small = x * _poly(x * x, _ERF_T)
    return jnp.where(abs_x < 1.0, 1.0 - erf_small, erfc_large)


def _body(alt_ref, cent_ref, ls_ref, dirs_ref, mu_ref, lsig_ref, llam_ref,
          als_ref, w_ref, seg_ref, logits_ref, lks_ref, acc_ref, *, nblocks,
          blk, num_segments):
    i = pl.program_id(0)
    k, e = dirs_ref.shape
    x = alt_ref[...]
    cent = cent_ref[...]
    dirs = dirs_ref[...]
    unit = dirs * jax.lax.rsqrt(jnp.sum(dirs * dirs, axis=1, keepdims=True))

    proj = jnp.concatenate([unit, cent], axis=0)
    m1 = jax.lax.dot_general(proj, x, (((1,), (1,)), ((), ())),
                             preferred_element_type=jnp.float32,
                             precision=_HIGHEST)
    sq = jax.lax.dot_general(jnp.ones((1, e), jnp.float32), x * x,
                             (((1,), (1,)), ((), ())),
                             preferred_element_type=jnp.float32,
                             precision=_HIGHEST)

    cu = jnp.sum(unit * cent, axis=1, keepdims=True)
    c2 = jnp.sum(cent * cent, axis=1, keepdims=True)
    norm2 = sq - 2.0 * m1[k:k + 1] + c2
    dot = m1[:k] - cu

    ls = ls_ref[0, 0]
    nonart = -float(e) * ls - norm2 * (0.5 * jnp.exp(-2.0 * ls))

    als = als_ref[...]
    orth2 = norm2 - dot * dot
    orth = -float(e - 1) * als - orth2 * (0.5 * jnp.exp(-2.0 * als))

    lsig = lsig_ref[...]
    llam = llam_ref[...]
    mu = mu_ref[...]
    var = jnp.exp(2.0 * lsig)
    lam = jnp.exp(llam)
    a = mu + lam * var
    z = (a - dot) / jnp.sqrt(2.0 * var)
    par = (llam - _LOG_TWO + jnp.log(_erfc(z))
           + 0.5 * lam * (mu + a - 2.0 * dot))

    vals = jnp.concatenate([nonart, orth + par], axis=0)

    seg = seg_ref[0]
    iota = jax.lax.broadcasted_iota(jnp.int32, (num_segments, blk), 0)
    onehot = (iota == seg).astype(jnp.float32)
    partial = jax.lax.dot_general(onehot, vals, (((1,), (1,)), ((), ())),
                                  preferred_element_type=jnp.float32,
                                  precision=_HIGHEST)

    @pl.when(i == 0)
    def _():
        acc_ref[...] = jnp.zeros_like(acc_ref)

    acc_ref[...] += partial

    @pl.when(i == nblocks - 1)
    def _():
        acc = acc_ref[...]
        wv = w_ref[...]
        m = jnp.max(wv, axis=1, keepdims=True)
        lw = wv - (m + jnp.log(jnp.sum(jnp.exp(wv - m), axis=1,
                                       keepdims=True)))
        art_bk = acc[:, 1:] + lw
        lks_ref[...] = jnp.concatenate([acc[:, :1], art_bk], axis=1)
        m2 = jnp.max(art_bk, axis=1, keepdims=True)
        alse = m2 + jnp.log(jnp.sum(jnp.exp(art_bk - m2), axis=1,
                                    keepdims=True))
        logits_ref[...] = alse - acc[:, :1]


def kernel(alt_flat, nonartifact_centroid_e, nonartifact_log_stdev,
           artifact_directions_ke, mu_k, log_sigma_k, log_lambda_k,
           artifact_log_stdev_k, cluster_weights_pre_softmax_k, segment_ids):
    r, e = alt_flat.shape
    k = mu_k.shape[0]
    b = 16
    blk = 1024
    nblocks = r // blk

    seg3 = segment_ids.reshape(nblocks, 1, blk)
    cent = nonartifact_centroid_e.reshape(1, e)
    ls = nonartifact_log_stdev.reshape(1, 1)
    mu = mu_k.reshape(k, 1)
    lsig = log_sigma_k.reshape(k, 1)
    llam = log_lambda_k.reshape(k, 1)
    als = artifact_log_stdev_k.reshape(k, 1)
    w = cluster_weights_pre_softmax_k.reshape(1, k)

    const = lambda *_: tuple(0 for _ in range(2))
    logits2d, lks = pl.pallas_call(
        functools.partial(_body, nblocks=nblocks, blk=blk, num_segments=b),
        grid=(nblocks,),
        in_specs=[
            pl.BlockSpec((blk, e), lambda i: (i, 0)),
            pl.BlockSpec((1, e), const),
            pl.BlockSpec((1, 1), const),
            pl.BlockSpec((k, e), const),
            pl.BlockSpec((k, 1), const),
            pl.BlockSpec((k, 1), const),
            pl.BlockSpec((k, 1), const),
            pl.BlockSpec((k, 1), const),
            pl.BlockSpec((1, k), const),
            pl.BlockSpec((1, 1, blk), lambda i: (i, 0, 0)),
        ],
        out_specs=[
            pl.BlockSpec((b, 1), const),
            pl.BlockSpec((b, k + 1), const),
        ],
        out_shape=[
            jax.ShapeDtypeStruct((b, 1), jnp.float32),
            jax.ShapeDtypeStruct((b, k + 1), jnp.float32),
        ],
        scratch_shapes=[pltpu.VMEM((b, k + 1), jnp.float32)],
        compiler_params=pltpu.CompilerParams(
            dimension_semantics=("arbitrary",)),
    )(alt_flat, cent, ls, artifact_directions_ke, mu, lsig, llam, als, w,
      seg3)
    return (logits2d.reshape(b), lks)
